# baseline (speedup 1.0000x reference)
import functools

import jax
import jax.numpy as jnp
from jax import lax
from jax.experimental import pallas as pl
from jax.experimental.pallas import tpu as pltpu
from jax.experimental.pallas import tpu_sc as plsc

NUM_TYPES = 8
D = 1024
B = 8192
BM = 512
B_PAD = B + NUM_TYPES * BM
NBLK_PAD = B_PAD // BM
MAX_WORK = NBLK_PAD
SUB = 64
LANES = 128
NW = 32
ROWS_PER_W = B // NW
CHUNK = 64
NCH = ROWS_PER_W // CHUNK


def _routing_body(types_ref, pos_ref, wl_ref):
    bf = jnp.bfloat16
    t = types_ref[...]
    R = NUM_TYPES * SUB
    m_rows = [(t == tt).astype(bf) for tt in range(NUM_TYPES)]
    M = jnp.concatenate(m_rows, axis=0)

    r512 = lax.broadcasted_iota(jnp.int32, (R, R), 0)
    c512 = lax.broadcasted_iota(jnp.int32, (R, R), 1)
    sl512 = (c512 < r512).astype(bf)
    r128 = lax.broadcasted_iota(jnp.int32, (LANES, LANES), 0)
    c128 = lax.broadcasted_iota(jnp.int32, (LANES, LANES), 1)
    upper_incl = (r128 <= c128).astype(bf)
    ones_l = jnp.ones((LANES, LANES), dtype=bf)

    f32 = jnp.float32
    dot = functools.partial(lax.dot, preferred_element_type=f32)
    full_rows = dot(sl512, M)
    incl = dot(full_rows.astype(bf), ones_l) + dot(M, upper_incl)

    rsel = lax.broadcasted_iota(jnp.int32, (NUM_TYPES, R), 0)
    csel = lax.broadcasted_iota(jnp.int32, (NUM_TYPES, R), 1)
    sel = (csel // SUB == rsel).astype(bf)
    counts = dot(sel, M)
    counts = dot(counts.astype(bf), ones_l)

    nblk = jnp.floor((counts + float(BM - 1)) * (1.0 / BM))
    pstart_rows = [jnp.zeros((1, LANES), dtype=f32)]
    acc = jnp.zeros((1, LANES), dtype=f32)
    gacc = jnp.zeros((1, LANES), dtype=f32)
    delta_rows = [jnp.zeros((1, LANES), dtype=f32)]
    for tt in range(1, NUM_TYPES):
        acc = acc + lax.slice_in_dim(nblk, tt - 1, tt, axis=0)
        gacc = gacc + lax.slice_in_dim(counts, tt - 1, tt, axis=0)
        pstart_rows.append(acc)
        delta_rows.append(acc * float(BM) - gacc)
    pblk_start = jnp.concatenate(pstart_rows, axis=0)

    pos_f = jnp.zeros((SUB, LANES), dtype=f32)
    for tt in range(NUM_TYPES):
        blk = lax.slice_in_dim(incl, tt * SUB, (tt + 1) * SUB, axis=0)
        pos_f = pos_f + m_rows[tt].astype(f32) * (delta_rows[tt] + blk - 1.0)
    pos_ref[...] = pos_f.astype(jnp.int32)

    total_items = jnp.sum(nblk, axis=0, keepdims=True)
    lmat = lax.broadcasted_iota(jnp.int32, (NUM_TYPES, LANES), 1).astype(f32)
    rowid = lax.broadcasted_iota(jnp.int32, (NUM_TYPES, LANES), 0).astype(f32)
    bel = jnp.logical_and(lmat >= pblk_start, lmat < pblk_start + nblk)

    def _sel(v):
        return jnp.sum(jnp.where(bel, v, 0.0), axis=0, keepdims=True)

    lane = lax.broadcasted_iota(jnp.int32, (1, LANES), 1).astype(f32)
    valid = lane < total_items
    g_last = jnp.max(jnp.where(nblk > 0.0, rowid, -1.0), axis=0, keepdims=True)
    rb = jnp.where(valid, lane, total_items - 1.0)
    tb = jnp.where(valid, _sel(rowid), g_last)
    vflag = jnp.where(valid, 1.0, 0.0)
    pad = jnp.zeros((NUM_TYPES - 3, LANES), dtype=f32)
    wl_ref[...] = jnp.concatenate([rb, tb, vflag, pad], axis=0).astype(
        jnp.int32)


def _routing(types2d):
    return pl.pallas_call(
        _routing_body,
        out_shape=(
            jax.ShapeDtypeStruct((SUB, LANES), jnp.int32),
            jax.ShapeDtypeStruct((NUM_TYPES, LANES), jnp.int32),
        ),
    )(types2d)


def _gmm_body(wl_ref, x_ref, w_ref, b_ref, out_ref):
    w = pl.program_id(0)

    @pl.when(wl_ref[2, w] == 1)
    def _compute():
        xb = x_ref[...].astype(jnp.bfloat16)
        wb = w_ref[0].astype(jnp.bfloat16)
        out_ref[...] = lax.dot_general(
            xb, wb, (((1,), (1,)), ((), ())),
            preferred_element_type=jnp.float32,
        ) + b_ref[0]


def _grouped_matmul(wl, x_sorted, W, b):
    grid_spec = pltpu.PrefetchScalarGridSpec(
        num_scalar_prefetch=1,
        grid=(MAX_WORK,),
        in_specs=[
            pl.BlockSpec((BM, D), lambda w, wl: (wl[0, w], 0)),
            pl.BlockSpec((1, D, D), lambda w, wl: (wl[1, w], 0, 0)),
            pl.BlockSpec((1, 1, D), lambda w, wl: (wl[1, w], 0, 0)),
        ],
        out_specs=pl.BlockSpec((BM, D), lambda w, wl: (wl[0, w], 0)),
    )
    return pl.pallas_call(
        _gmm_body,
        grid_spec=grid_spec,
        out_shape=jax.ShapeDtypeStruct((B_PAD, D), jnp.float32),
        compiler_params=pltpu.CompilerParams(
            dimension_semantics=("arbitrary",),
        ),
    )(wl, x_sorted, W, b.reshape(NUM_TYPES, 1, D))


def _sc_scatter_body(x_hbm, pos_hbm, out_hbm, idx_v, rows_v, sem):
    wid = lax.axis_index("s") * 2 + lax.axis_index("c")
    base = wid * ROWS_PER_W
    for k in range(NCH):
        off = base + k * CHUNK
        pltpu.sync_copy(pos_hbm.at[pl.ds(off, CHUNK)], idx_v)
        pltpu.sync_copy(x_hbm.at[pl.ds(off, CHUNK)], rows_v)
        pltpu.async_copy(rows_v, out_hbm.at[idx_v], sem).wait()


def _sc_gather_body(y_hbm, pos_hbm, out_hbm, idx_v, rows_v, sem):
    wid = lax.axis_index("s") * 2 + lax.axis_index("c")
    base = wid * ROWS_PER_W
    for k in range(NCH):
        off = base + k * CHUNK
        pltpu.sync_copy(pos_hbm.at[pl.ds(off, CHUNK)], idx_v)
        pltpu.async_copy(y_hbm.at[idx_v], rows_v, sem).wait()
        pltpu.sync_copy(rows_v, out_hbm.at[pl.ds(off, CHUNK)])


@functools.lru_cache(maxsize=None)
def _sc_kernels():
    mesh = plsc.VectorSubcoreMesh(
        core_axis_name="c", subcore_axis_name="s", num_cores=2, num_subcores=16
    )
    scratch = [
        pltpu.VMEM((CHUNK,), jnp.int32),
        pltpu.VMEM((CHUNK, D), jnp.float32),
        pltpu.SemaphoreType.DMA,
    ]
    scatter = pl.kernel(
        _sc_scatter_body,
        out_type=jax.ShapeDtypeStruct((B_PAD, D), jnp.float32),
        mesh=mesh,
        scratch_types=scratch,
    )
    gather = pl.kernel(
        _sc_gather_body,
        out_type=jax.ShapeDtypeStruct((B, D), jnp.float32),
        mesh=mesh,
        scratch_types=scratch,
    )
    return scatter, gather


def kernel(x, types, W, b):
    types2d = types.reshape(SUB, LANES)
    pos2d, wl = _routing(types2d)
    wlp = lax.slice(wl, (0, 0), (3, MAX_WORK))
    scatter_rows, gather_rows = _sc_kernels()
    pos_sc = pos2d.reshape(B)
    x_sorted = scatter_rows(x, pos_sc)
    y_sorted = _grouped_matmul(wlp, x_sorted, W, b)
    return gather_rows(y_sorted, pos_sc)


# speedup vs baseline: 1.0074x; 1.0074x over previous
import functools

import jax
import jax.numpy as jnp
from jax import lax
from jax.experimental import pallas as pl
from jax.experimental.pallas import tpu as pltpu
from jax.experimental.pallas import tpu_sc as plsc

NUM_TYPES = 8
D = 1024
B = 8192
BM = 1024
B_PAD = B + NUM_TYPES * BM
NBLK_PAD = B_PAD // BM
MAX_WORK = NBLK_PAD
SUB = 64
LANES = 128
NW = 32
ROWS_PER_W = B // NW
CHUNK = 64
NCH = ROWS_PER_W // CHUNK


def _routing_body(types_ref, pos_ref, wl_ref):
    bf = jnp.bfloat16
    t = types_ref[...]
    R = NUM_TYPES * SUB
    m_rows = [(t == tt).astype(bf) for tt in range(NUM_TYPES)]
    M = jnp.concatenate(m_rows, axis=0)

    r512 = lax.broadcasted_iota(jnp.int32, (R, R), 0)
    c512 = lax.broadcasted_iota(jnp.int32, (R, R), 1)
    sl512 = (c512 < r512).astype(bf)
    r128 = lax.broadcasted_iota(jnp.int32, (LANES, LANES), 0)
    c128 = lax.broadcasted_iota(jnp.int32, (LANES, LANES), 1)
    upper_incl = (r128 <= c128).astype(bf)
    ones_l = jnp.ones((LANES, LANES), dtype=bf)

    f32 = jnp.float32
    dot = functools.partial(lax.dot, preferred_element_type=f32)
    full_rows = dot(sl512, M)
    incl = dot(full_rows.astype(bf), ones_l) + dot(M, upper_incl)

    rsel = lax.broadcasted_iota(jnp.int32, (NUM_TYPES, R), 0)
    csel = lax.broadcasted_iota(jnp.int32, (NUM_TYPES, R), 1)
    sel = (csel // SUB == rsel).astype(bf)
    counts = dot(sel, M)
    counts = dot(counts.astype(bf), ones_l)

    nblk = jnp.floor((counts + float(BM - 1)) * (1.0 / BM))
    pstart_rows = [jnp.zeros((1, LANES), dtype=f32)]
    acc = jnp.zeros((1, LANES), dtype=f32)
    gacc = jnp.zeros((1, LANES), dtype=f32)
    delta_rows = [jnp.zeros((1, LANES), dtype=f32)]
    for tt in range(1, NUM_TYPES):
        acc = acc + lax.slice_in_dim(nblk, tt - 1, tt, axis=0)
        gacc = gacc + lax.slice_in_dim(counts, tt - 1, tt, axis=0)
        pstart_rows.append(acc)
        delta_rows.append(acc * float(BM) - gacc)
    pblk_start = jnp.concatenate(pstart_rows, axis=0)

    pos_f = jnp.zeros((SUB, LANES), dtype=f32)
    for tt in range(NUM_TYPES):
        blk = lax.slice_in_dim(incl, tt * SUB, (tt + 1) * SUB, axis=0)
        pos_f = pos_f + m_rows[tt].astype(f32) * (delta_rows[tt] + blk - 1.0)
    pos_ref[...] = pos_f.astype(jnp.int32)

    total_items = jnp.sum(nblk, axis=0, keepdims=True)
    lmat = lax.broadcasted_iota(jnp.int32, (NUM_TYPES, LANES), 1).astype(f32)
    rowid = lax.broadcasted_iota(jnp.int32, (NUM_TYPES, LANES), 0).astype(f32)
    bel = jnp.logical_and(lmat >= pblk_start, lmat < pblk_start + nblk)

    def _sel(v):
        return jnp.sum(jnp.where(bel, v, 0.0), axis=0, keepdims=True)

    lane = lax.broadcasted_iota(jnp.int32, (1, LANES), 1).astype(f32)
    valid = lane < total_items
    g_last = jnp.max(jnp.where(nblk > 0.0, rowid, -1.0), axis=0, keepdims=True)
    rb = jnp.where(valid, lane, total_items - 1.0)
    tb = jnp.where(valid, _sel(rowid), g_last)
    vflag = jnp.where(valid, 1.0, 0.0)
    pad = jnp.zeros((NUM_TYPES - 3, LANES), dtype=f32)
    wl_ref[...] = jnp.concatenate([rb, tb, vflag, pad], axis=0).astype(
        jnp.int32)


def _routing(types2d):
    return pl.pallas_call(
        _routing_body,
        out_shape=(
            jax.ShapeDtypeStruct((SUB, LANES), jnp.int32),
            jax.ShapeDtypeStruct((NUM_TYPES, LANES), jnp.int32),
        ),
    )(types2d)


def _gmm_body(wl_ref, x_ref, w_ref, b_ref, out_ref):
    w = pl.program_id(0)

    @pl.when(wl_ref[2, w] == 1)
    def _compute():
        xb = x_ref[...].astype(jnp.bfloat16)
        wb = w_ref[0].astype(jnp.bfloat16)
        out_ref[...] = lax.dot_general(
            xb, wb, (((1,), (1,)), ((), ())),
            preferred_element_type=jnp.float32,
        ) + b_ref[0]


def _grouped_matmul(wl, x_sorted, W, b):
    grid_spec = pltpu.PrefetchScalarGridSpec(
        num_scalar_prefetch=1,
        grid=(MAX_WORK,),
        in_specs=[
            pl.BlockSpec((BM, D), lambda w, wl: (wl[0, w], 0)),
            pl.BlockSpec((1, D, D), lambda w, wl: (wl[1, w], 0, 0)),
            pl.BlockSpec((1, 1, D), lambda w, wl: (wl[1, w], 0, 0)),
        ],
        out_specs=pl.BlockSpec((BM, D), lambda w, wl: (wl[0, w], 0)),
    )
    return pl.pallas_call(
        _gmm_body,
        grid_spec=grid_spec,
        out_shape=jax.ShapeDtypeStruct((B_PAD, D), jnp.float32),
        compiler_params=pltpu.CompilerParams(
            dimension_semantics=("arbitrary",),
        ),
    )(wl, x_sorted, W, b.reshape(NUM_TYPES, 1, D))


def _sc_scatter_body(x_hbm, pos_hbm, out_hbm, idx_v, rows_v, sem):
    wid = lax.axis_index("s") * 2 + lax.axis_index("c")
    base = wid * ROWS_PER_W
    for k in range(NCH):
        off = base + k * CHUNK
        pltpu.sync_copy(pos_hbm.at[pl.ds(off, CHUNK)], idx_v)
        pltpu.sync_copy(x_hbm.at[pl.ds(off, CHUNK)], rows_v)
        pltpu.async_copy(rows_v, out_hbm.at[idx_v], sem).wait()


def _sc_gather_body(y_hbm, pos_hbm, out_hbm, idx_v, rows_v, sem):
    wid = lax.axis_index("s") * 2 + lax.axis_index("c")
    base = wid * ROWS_PER_W
    for k in range(NCH):
        off = base + k * CHUNK
        pltpu.sync_copy(pos_hbm.at[pl.ds(off, CHUNK)], idx_v)
        pltpu.async_copy(y_hbm.at[idx_v], rows_v, sem).wait()
        pltpu.sync_copy(rows_v, out_hbm.at[pl.ds(off, CHUNK)])


@functools.lru_cache(maxsize=None)
def _sc_kernels():
    mesh = plsc.VectorSubcoreMesh(
        core_axis_name="c", subcore_axis_name="s", num_cores=2, num_subcores=16
    )
    scratch = [
        pltpu.VMEM((CHUNK,), jnp.int32),
        pltpu.VMEM((CHUNK, D), jnp.float32),
        pltpu.SemaphoreType.DMA,
    ]
    scatter = pl.kernel(
        _sc_scatter_body,
        out_type=jax.ShapeDtypeStruct((B_PAD, D), jnp.float32),
        mesh=mesh,
        scratch_types=scratch,
    )
    gather = pl.kernel(
        _sc_gather_body,
        out_type=jax.ShapeDtypeStruct((B, D), jnp.float32),
        mesh=mesh,
        scratch_types=scratch,
    )
    return scatter, gather


def kernel(x, types, W, b):
    types2d = types.reshape(SUB, LANES)
    pos2d, wl = _routing(types2d)
    wlp = lax.slice(wl, (0, 0), (3, MAX_WORK))
    scatter_rows, gather_rows = _sc_kernels()
    pos_sc = pos2d.reshape(B)
    x_sorted = scatter_rows(x, pos_sc)
    y_sorted = _grouped_matmul(wlp, x_sorted, W, b)
    return gather_rows(y_sorted, pos_sc)
